# baseline (speedup 1.0000x reference)
import functools

import jax
import jax.numpy as jnp
from jax import lax
from jax.experimental import pallas as pl
from jax.experimental.pallas import tpu as pltpu
from jax.experimental.pallas import tpu_sc as plsc

_EMB_DIM = 128
_NUM_SEG = 26
_PERM = tuple(range(_NUM_SEG - 1, -1, -1))
_BATCH = 16384
_ROW = _NUM_SEG * _EMB_DIM


def _permute_sc(pooled_embs):
    info = plsc.get_sparse_core_info()
    num_workers = info.num_cores * info.num_subcores
    rows_per_w = _BATCH // num_workers
    mesh = plsc.VectorSubcoreMesh(core_axis_name="c", subcore_axis_name="s")

    @functools.partial(
        pl.kernel,
        mesh=mesh,
        out_type=jax.ShapeDtypeStruct((_BATCH, _ROW), jnp.float32),
        scratch_types=[pltpu.SemaphoreType.DMA],
    )
    def k(in_hbm, out_hbm, sem):
        wid = lax.axis_index("s") * info.num_cores + lax.axis_index("c")
        base = wid * rows_per_w
        copies = []
        for j in range(_NUM_SEG):
            src = _PERM[j]
            c = pltpu.make_async_copy(
                in_hbm.at[pl.ds(base, rows_per_w), pl.ds(src * _EMB_DIM, _EMB_DIM)],
                out_hbm.at[pl.ds(base, rows_per_w), pl.ds(j * _EMB_DIM, _EMB_DIM)],
                sem,
            )
            c.start()
            copies.append(c)
        for c in copies:
            c.wait()

    return k(pooled_embs)


def kernel(pooled_embs):
    return _permute_sc(pooled_embs)


# speedup vs baseline: 9.9845x; 9.9845x over previous
import functools

import jax
import jax.numpy as jnp
import numpy as np
from jax import lax
from jax.experimental import pallas as pl
from jax.experimental.pallas import tpu as pltpu
from jax.experimental.pallas import tpu_sc as plsc

_EMB_DIM = 128
_NUM_SEG = 26
_PERM = np.arange(_NUM_SEG - 1, -1, -1)
_BATCH = 16384
_CHUNK = 128


def _build_index_table(num_workers: int) -> np.ndarray:
    rows_per_w = _BATCH // num_workers
    segrows_per_w = rows_per_w * _NUM_SEG
    r = np.arange(num_workers * segrows_per_w, dtype=np.int64)
    src = (r // _NUM_SEG) * _NUM_SEG + _PERM[r % _NUM_SEG]
    n_chunks = segrows_per_w // _CHUNK
    return src.astype(np.int32).reshape(num_workers, n_chunks, _CHUNK)


def _permute_sc(pooled_embs):
    info = plsc.get_sparse_core_info()
    num_workers = info.num_cores * info.num_subcores
    rows_per_w = _BATCH // num_workers
    segrows_per_w = rows_per_w * _NUM_SEG
    n_chunks = segrows_per_w // _CHUNK
    total_segrows = _BATCH * _NUM_SEG

    idx_table = jnp.asarray(_build_index_table(num_workers))
    in_view = pooled_embs.reshape(total_segrows, _EMB_DIM)
    mesh = plsc.VectorSubcoreMesh(core_axis_name="c", subcore_axis_name="s")

    @functools.partial(
        pl.kernel,
        mesh=mesh,
        out_type=jax.ShapeDtypeStruct((total_segrows, _EMB_DIM), jnp.float32),
        scratch_types=[
            pltpu.VMEM((n_chunks, _CHUNK), jnp.int32),
            pltpu.VMEM((_CHUNK, _EMB_DIM), jnp.float32),
            pltpu.SemaphoreType.DMA,
            pltpu.SemaphoreType.DMA,
        ],
    )
    def k(in_hbm, idx_hbm, out_hbm, idx_v, buf, gsem, ssem):
        wid = lax.axis_index("s") * info.num_cores + lax.axis_index("c")
        out_base = wid * segrows_per_w
        pltpu.sync_copy(idx_hbm.at[wid], idx_v)

        def body(c, carry):
            g = pltpu.make_async_copy(in_hbm.at[idx_v.at[c]], buf, gsem)
            g.start()
            g.wait()
            s = pltpu.make_async_copy(
                buf, out_hbm.at[pl.ds(out_base + c * _CHUNK, _CHUNK)], ssem
            )
            s.start()
            s.wait()
            return carry

        lax.fori_loop(0, n_chunks, body, 0)

    out = k(in_view, idx_table)
    return out.reshape(_BATCH, _NUM_SEG * _EMB_DIM)


def kernel(pooled_embs):
    return _permute_sc(pooled_embs)


# speedup vs baseline: 11.2015x; 1.1219x over previous
import functools

import jax
import jax.numpy as jnp
import numpy as np
from jax import lax
from jax.experimental import pallas as pl
from jax.experimental.pallas import tpu as pltpu
from jax.experimental.pallas import tpu_sc as plsc

_EMB_DIM = 128
_NUM_SEG = 26
_PERM = np.arange(_NUM_SEG - 1, -1, -1)
_BATCH = 16384
_IDXW = 128
_CHUNK_IDX_ROWS = 3


def _build_index_table(num_workers: int) -> np.ndarray:
    segrows_per_w = (_BATCH // num_workers) * _NUM_SEG
    r = np.arange(num_workers * segrows_per_w, dtype=np.int64)
    src = (r // _NUM_SEG) * _NUM_SEG + _PERM[r % _NUM_SEG]
    return src.astype(np.int32).reshape(num_workers, segrows_per_w)


def _permute_sc(pooled_embs):
    info = plsc.get_sparse_core_info()
    num_workers = info.num_cores * info.num_subcores
    segrows_per_w = (_BATCH // num_workers) * _NUM_SEG
    n_idx_rows = segrows_per_w // _IDXW
    total_segrows = _BATCH * _NUM_SEG
    buf_rows = _CHUNK_IDX_ROWS * _IDXW

    chunks = []
    pos = 0
    while pos < n_idx_rows:
        n = min(_CHUNK_IDX_ROWS, n_idx_rows - pos)
        chunks.append((pos, n))
        pos += n

    idx_table = jnp.asarray(_build_index_table(num_workers))
    in_view = pooled_embs.reshape(total_segrows, _EMB_DIM)
    mesh = plsc.VectorSubcoreMesh(core_axis_name="c", subcore_axis_name="s")

    @functools.partial(
        pl.kernel,
        mesh=mesh,
        out_type=jax.ShapeDtypeStruct((total_segrows, _EMB_DIM), jnp.float32),
        scratch_types=[
            pltpu.VMEM((segrows_per_w,), jnp.int32),
            pltpu.VMEM((buf_rows, _EMB_DIM), jnp.float32),
            pltpu.VMEM((buf_rows, _EMB_DIM), jnp.float32),
            pltpu.SemaphoreType.DMA,
            pltpu.SemaphoreType.DMA,
            pltpu.SemaphoreType.DMA,
            pltpu.SemaphoreType.DMA,
        ],
    )
    def k(in_hbm, idx_hbm, out_hbm, idx_v, buf_a, buf_b, ga, gb, sa, sb):
        wid = lax.axis_index("s") * info.num_cores + lax.axis_index("c")
        out_base = wid * segrows_per_w
        pltpu.sync_copy(idx_hbm.at[wid], idx_v)

        bufs = (buf_a, buf_b)
        gsems = (ga, gb)
        ssems = (sa, sb)

        def gather(t):
            start, n = chunks[t]
            c = pltpu.make_async_copy(
                in_hbm.at[idx_v.at[pl.ds(start * _IDXW, n * _IDXW)]],
                bufs[t % 2].at[pl.ds(0, n * _IDXW)],
                gsems[t % 2],
            )
            c.start()
            return c

        def scatter(t):
            start, n = chunks[t]
            c = pltpu.make_async_copy(
                bufs[t % 2].at[pl.ds(0, n * _IDXW)],
                out_hbm.at[pl.ds(out_base + start * _IDXW, n * _IDXW)],
                ssems[t % 2],
            )
            c.start()
            return c

        n_chunks = len(chunks)
        pending_g = gather(0)
        pending_s = [None, None]
        for t in range(n_chunks):
            pending_g.wait()
            if pending_s[(t + 1) % 2] is not None:
                pending_s[(t + 1) % 2].wait()
                pending_s[(t + 1) % 2] = None
            if t + 1 < n_chunks:
                pending_g = gather(t + 1)
            pending_s[t % 2] = scatter(t)
        for s in pending_s:
            if s is not None:
                s.wait()

    out = k(in_view, idx_table)
    return out.reshape(_BATCH, _NUM_SEG * _EMB_DIM)


def kernel(pooled_embs):
    return _permute_sc(pooled_embs)


# speedup vs baseline: 31.1188x; 2.7781x over previous
import functools

import jax
import jax.numpy as jnp
from jax import lax
from jax.experimental import pallas as pl
from jax.experimental.pallas import tpu as pltpu
from jax.experimental.pallas import tpu_sc as plsc

_EMB_DIM = 128
_NUM_SEG = 26
_PERM = tuple(range(_NUM_SEG - 1, -1, -1))
_BATCH = 16384
_ROW = _NUM_SEG * _EMB_DIM
_CHUNK_ROWS = 128


def _permute_sc(pooled_embs):
    info = plsc.get_sparse_core_info()
    num_workers = info.num_cores * info.num_subcores
    rows_per_w = _BATCH // num_workers
    n_rchunks = rows_per_w // _CHUNK_ROWS
    mesh = plsc.VectorSubcoreMesh(core_axis_name="c", subcore_axis_name="s")

    @functools.partial(
        pl.kernel,
        mesh=mesh,
        out_type=jax.ShapeDtypeStruct((_BATCH, _ROW), jnp.float32),
        scratch_types=[
            pltpu.VMEM((_CHUNK_ROWS, _EMB_DIM), jnp.float32),
            pltpu.VMEM((_CHUNK_ROWS, _EMB_DIM), jnp.float32),
            pltpu.SemaphoreType.DMA,
            pltpu.SemaphoreType.DMA,
            pltpu.SemaphoreType.DMA,
            pltpu.SemaphoreType.DMA,
        ],
    )
    def k(in_hbm, out_hbm, buf_a, buf_b, ga, gb, sa, sb):
        wid = lax.axis_index("s") * info.num_cores + lax.axis_index("c")
        row_base = wid * rows_per_w

        bufs = (buf_a, buf_b)
        gsems = (ga, gb)
        ssems = (sa, sb)
        steps = [
            (j, c) for j in range(_NUM_SEG) for c in range(n_rchunks)
        ]

        def gather(t):
            j, c = steps[t]
            src = _PERM[j]
            h = pltpu.make_async_copy(
                in_hbm.at[
                    pl.ds(row_base + c * _CHUNK_ROWS, _CHUNK_ROWS),
                    pl.ds(src * _EMB_DIM, _EMB_DIM),
                ],
                bufs[t % 2],
                gsems[t % 2],
            )
            h.start()
            return h

        def scatter(t):
            j, c = steps[t]
            h = pltpu.make_async_copy(
                bufs[t % 2],
                out_hbm.at[
                    pl.ds(row_base + c * _CHUNK_ROWS, _CHUNK_ROWS),
                    pl.ds(j * _EMB_DIM, _EMB_DIM),
                ],
                ssems[t % 2],
            )
            h.start()
            return h

        n_steps = len(steps)
        pending_g = gather(0)
        pending_s = [None, None]
        for t in range(n_steps):
            pending_g.wait()
            other = (t + 1) % 2
            if pending_s[other] is not None:
                pending_s[other].wait()
                pending_s[other] = None
            if t + 1 < n_steps:
                pending_g = gather(t + 1)
            pending_s[t % 2] = scatter(t)
        for s in pending_s:
            if s is not None:
                s.wait()

    return k(pooled_embs)


def kernel(pooled_embs):
    return _permute_sc(pooled_embs)


# speedup vs baseline: 38.1811x; 1.2269x over previous
import functools

import jax
import jax.numpy as jnp
from jax import lax
from jax.experimental import pallas as pl
from jax.experimental.pallas import tpu as pltpu
from jax.experimental.pallas import tpu_sc as plsc

_EMB_DIM = 128
_NUM_SEG = 26
_PERM = tuple(range(_NUM_SEG - 1, -1, -1))
_BATCH = 16384
_ROW = _NUM_SEG * _EMB_DIM
_CHUNK_ROWS = 128


def _permute_sc(pooled_embs):
    info = plsc.get_sparse_core_info()
    num_workers = info.num_cores * info.num_subcores
    rows_per_w = _BATCH // num_workers
    n_rchunks = rows_per_w // _CHUNK_ROWS
    mesh = plsc.VectorSubcoreMesh(core_axis_name="c", subcore_axis_name="s")

    @functools.partial(
        pl.kernel,
        mesh=mesh,
        out_type=jax.ShapeDtypeStruct((_BATCH, _ROW), jnp.float32),
        scratch_types=[
            pltpu.VMEM((_CHUNK_ROWS, _EMB_DIM), jnp.float32),
            pltpu.VMEM((_CHUNK_ROWS, _EMB_DIM), jnp.float32),
            pltpu.VMEM((_CHUNK_ROWS, _EMB_DIM), jnp.float32),
            pltpu.VMEM((_CHUNK_ROWS, _EMB_DIM), jnp.float32),
            pltpu.SemaphoreType.DMA,
            pltpu.SemaphoreType.DMA,
            pltpu.SemaphoreType.DMA,
            pltpu.SemaphoreType.DMA,
            pltpu.SemaphoreType.DMA,
            pltpu.SemaphoreType.DMA,
            pltpu.SemaphoreType.DMA,
            pltpu.SemaphoreType.DMA,
        ],
    )
    def k(in_hbm, out_hbm, b0, b1, b2, b3, g0, g1, g2, g3, s0, s1, s2, s3):
        wid = lax.axis_index("s") * info.num_cores + lax.axis_index("c")
        row_base = wid * rows_per_w

        nbuf = 4
        bufs = (b0, b1, b2, b3)
        gsems = (g0, g1, g2, g3)
        ssems = (s0, s1, s2, s3)
        steps = [
            (j, c) for j in range(_NUM_SEG) for c in range(n_rchunks)
        ]
        n_steps = len(steps)

        def gather(t):
            j, c = steps[t]
            src = _PERM[j]
            h = pltpu.make_async_copy(
                in_hbm.at[
                    pl.ds(row_base + c * _CHUNK_ROWS, _CHUNK_ROWS),
                    pl.ds(src * _EMB_DIM, _EMB_DIM),
                ],
                bufs[t % nbuf],
                gsems[t % nbuf],
            )
            h.start()
            return h

        def scatter(t):
            j, c = steps[t]
            h = pltpu.make_async_copy(
                bufs[t % nbuf],
                out_hbm.at[
                    pl.ds(row_base + c * _CHUNK_ROWS, _CHUNK_ROWS),
                    pl.ds(j * _EMB_DIM, _EMB_DIM),
                ],
                ssems[t % nbuf],
            )
            h.start()
            return h

        g_pend = {}
        s_pend = {}
        g_pend[0] = gather(0)
        g_pend[1] = gather(1)
        for t in range(n_steps):
            g_pend.pop(t).wait()
            s_pend[t] = scatter(t)
            u = t + 2
            if u < n_steps:
                if u - nbuf in s_pend:
                    s_pend.pop(u - nbuf).wait()
                g_pend[u] = gather(u)
        for t in sorted(s_pend):
            s_pend.pop(t).wait()

    return k(pooled_embs)


def kernel(pooled_embs):
    return _permute_sc(pooled_embs)
